# Initial kernel scaffold; baseline (speedup 1.0000x reference)
import functools

import jax
import jax.numpy as jnp
from jax import lax
from jax.experimental import pallas as pl
from jax.experimental.pallas import tpu as pltpu
from jax.experimental.pallas import tpu_sc as plsc

N = 10000
E = 320000
D = 128
ALPHA = 0.2

NC = 2
NS = 16
NW = NC * NS
EPT = E // NW
CHUNK = 80
NCHUNK = EPT // CHUNK
ROWS_PER_SUB = N // NS
SLAB = 125
ACC_W = 144

_f32 = jnp.float32


def _prep_body(x_ref, wfc_ref, wattn_ref, h_ref, asrc_ref, adst_ref):
    xb = x_ref[...]
    h = lax.dot_general(xb, wfc_ref[...], (((1,), (1,)), ((), ())),
                        preferred_element_type=_f32)
    wa = wattn_ref[...]
    a1 = wa[:, :D]
    a2 = wa[:, D:]
    h_ref[...] = h
    asrc_ref[...] = lax.dot_general(a1, h, (((1,), (1,)), ((), ())),
                                    preferred_element_type=_f32)
    adst_ref[...] = lax.dot_general(a2, h, (((1,), (1,)), ((), ())),
                                    preferred_element_type=_f32)


_PREP_B = 400


def _prep(x, W_fc, W_attn):
    grid = N // _PREP_B
    return pl.pallas_call(
        _prep_body,
        grid=(grid,),
        in_specs=[
            pl.BlockSpec((_PREP_B, D), lambda i: (i, 0)),
            pl.BlockSpec((D, D), lambda i: (0, 0)),
            pl.BlockSpec((1, 2 * D), lambda i: (0, 0)),
        ],
        out_specs=[
            pl.BlockSpec((_PREP_B, D), lambda i: (i, 0)),
            pl.BlockSpec((1, _PREP_B), lambda i: (0, i)),
            pl.BlockSpec((1, _PREP_B), lambda i: (0, i)),
        ],
        out_shape=[
            jax.ShapeDtypeStruct((N, D), _f32),
            jax.ShapeDtypeStruct((1, N), _f32),
            jax.ShapeDtypeStruct((1, N), _f32),
        ],
    )(x, W_fc, W_attn)


def _edge_body(h_hbm, asrc_hbm, adst_hbm, src_hbm, dst_hbm, out_hbm,
               src2d, dst2d, asrc_t, adst_t, gbuf, sbuf, bounce, wtmp, acc):
    cid = lax.axis_index("c")
    sid = lax.axis_index("s")
    wid = cid * NS + sid

    zero16 = jnp.zeros((16,), _f32)

    pltpu.sync_copy(src_hbm.at[wid], src2d)
    pltpu.sync_copy(dst_hbm.at[wid], dst2d)
    pltpu.sync_copy(asrc_hbm.at[0], asrc_t)
    pltpu.sync_copy(adst_hbm.at[0], adst_t)

    @pl.loop(0, SLAB)
    def _zb(i):
        for j in range(0, ACC_W, 16):
            bounce[i, pl.ds(j, 16)] = zero16

    @pl.loop(0, CHUNK)
    def _zs(i):
        for j in range(0, ACC_W, 16):
            sbuf[i, pl.ds(j, 16)] = zero16

    @pl.loop(0, 5)
    def _zacc(j):
        pltpu.sync_copy(bounce, acc.at[pl.ds(sid * ROWS_PER_SUB + j * SLAB, SLAB)])

    plsc.subcore_barrier()

    iota16 = lax.iota(jnp.int32, 16)
    col_w = jnp.full((16,), D, jnp.int32)

    @pl.loop(0, NCHUNK)
    def _chunk(c):
        pltpu.sync_copy(h_hbm.at[dst2d.at[c]], gbuf)

        for g in range(CHUNK // 16):
            src16 = src2d[c, pl.ds(g * 16, 16)]
            dst16 = dst2d[c, pl.ds(g * 16, 16)]
            a_s = plsc.load_gather(asrc_t, [src16])
            a_d = plsc.load_gather(adst_t, [dst16])
            s = a_s + a_d
            leaky = jnp.where(s > 0, s, ALPHA * s)
            w16 = jnp.exp(-leaky)
            rowids = iota16 + (g * 16)
            plsc.store_scatter(sbuf, [rowids, col_w], w16)
            wtmp[...] = w16
            for r in range(16):
                row = g * 16 + r
                wr = plsc.load_gather(wtmp, [jnp.full((16,), r, jnp.int32)])
                for j in range(0, D, 16):
                    sbuf[row, pl.ds(j, 16)] = gbuf[row, pl.ds(j, 16)] * wr

        pltpu.sync_copy(sbuf, acc.at[src2d.at[c]], add=True)

    plsc.subcore_barrier()

    @pl.loop(0, 5)
    def _rb(j):
        base = sid * ROWS_PER_SUB + j * SLAB
        pltpu.sync_copy(acc.at[pl.ds(base, SLAB)], bounce)
        pltpu.sync_copy(bounce, out_hbm.at[cid, pl.ds(base, SLAB)])


_edge_kernel = pl.kernel(
    _edge_body,
    out_type=jax.ShapeDtypeStruct((NC, N, ACC_W), _f32),
    mesh=plsc.VectorSubcoreMesh(core_axis_name="c", subcore_axis_name="s"),
    scratch_types=[
        pltpu.VMEM((NCHUNK, CHUNK), jnp.int32),
        pltpu.VMEM((NCHUNK, CHUNK), jnp.int32),
        pltpu.VMEM((N,), _f32),
        pltpu.VMEM((N,), _f32),
        pltpu.VMEM((CHUNK, D), _f32),
        pltpu.VMEM((CHUNK, ACC_W), _f32),
        pltpu.VMEM((SLAB, ACC_W), _f32),
        pltpu.VMEM((16,), _f32),
        pltpu.VMEM_SHARED((N, ACC_W), _f32),
    ],
)


def _final_body(acc_ref, nn_ref, o_ref):
    a = acc_ref[0] + acc_ref[1]
    num = a[:, :D]
    den = a[:, D:D + 1]
    y = num / den * nn_ref[...]
    o_ref[...] = jnp.where(y > 0, y, jnp.expm1(jnp.minimum(y, 0.0)))


_FIN_B = 400


def _final(acc, n_norm):
    grid = N // _FIN_B
    return pl.pallas_call(
        _final_body,
        grid=(grid,),
        in_specs=[
            pl.BlockSpec((NC, _FIN_B, ACC_W), lambda i: (0, i, 0)),
            pl.BlockSpec((_FIN_B, 1), lambda i: (i, 0)),
        ],
        out_specs=pl.BlockSpec((_FIN_B, D), lambda i: (i, 0)),
        out_shape=jax.ShapeDtypeStruct((N, D), _f32),
    )(acc, n_norm)


def kernel(x, edge_index, n_norm, W_fc, W_attn):
    h, asrc, adst = _prep(x, W_fc, W_attn)
    src = edge_index[0].reshape(NW, NCHUNK, CHUNK)
    dst = edge_index[1].reshape(NW, NCHUNK, CHUNK)
    acc = _edge_kernel(h, asrc, adst, src, dst)
    return _final(acc, n_norm)


# speedup vs baseline: 5.5749x; 5.5749x over previous
import functools

import jax
import jax.numpy as jnp
from jax import lax
from jax.experimental import pallas as pl
from jax.experimental.pallas import tpu as pltpu
from jax.experimental.pallas import tpu_sc as plsc

N = 10000
E = 320000
D = 128
ALPHA = 0.2

NC = 2
NS = 16
NW = NC * NS
EPT = E // NW
CHUNK = 80
NCHUNK = EPT // CHUNK
ROWS_PER_SUB = N // NS
SLAB = 25
ACC_W = 144

_f32 = jnp.float32


def _splat_lane(v, lane):
    dn = lax.GatherDimensionNumbers(
        offset_dims=(), collapsed_slice_dims=(0,), start_index_map=(0,))
    idx = jnp.full((16, 1), lane, jnp.int32)
    return lax.gather(v, idx, dn, (1,),
                      mode=lax.GatherScatterMode.PROMISE_IN_BOUNDS)


def _prep_body(x_ref, wfc_ref, wattn_ref, h_ref, asrc_ref, adst_ref):
    xb = x_ref[...]
    h = lax.dot_general(xb, wfc_ref[...], (((1,), (1,)), ((), ())),
                        preferred_element_type=_f32)
    wa = wattn_ref[...]
    a1 = wa[:, :D]
    a2 = wa[:, D:]
    h_ref[...] = h
    asrc_ref[...] = jnp.sum(h * a1, axis=1, keepdims=True)
    adst_ref[...] = jnp.sum(h * a2, axis=1, keepdims=True)


_PREP_B = 400


def _prep(x, W_fc, W_attn):
    grid = N // _PREP_B
    return pl.pallas_call(
        _prep_body,
        grid=(grid,),
        in_specs=[
            pl.BlockSpec((_PREP_B, D), lambda i: (i, 0)),
            pl.BlockSpec((D, D), lambda i: (0, 0)),
            pl.BlockSpec((1, 2 * D), lambda i: (0, 0)),
        ],
        out_specs=[
            pl.BlockSpec((_PREP_B, D), lambda i: (i, 0)),
            pl.BlockSpec((_PREP_B, 1), lambda i: (i, 0)),
            pl.BlockSpec((_PREP_B, 1), lambda i: (i, 0)),
        ],
        out_shape=[
            jax.ShapeDtypeStruct((N, D), _f32),
            jax.ShapeDtypeStruct((N, 1), _f32),
            jax.ShapeDtypeStruct((N, 1), _f32),
        ],
    )(x, W_fc, W_attn)


WIN = 25
NWIN = NCHUNK // WIN


def _edge_body(h_hbm, asrc_hbm, adst_hbm, src_hbm, dst_hbm, out_hbm,
               srcw, dstw, asbuf, adbuf, gbuf, sbuf, bounce, acc):
    cid = lax.axis_index("c")
    sid = lax.axis_index("s")
    wid = cid * NS + sid

    zero16 = jnp.zeros((16,), _f32)

    @pl.loop(0, SLAB)
    def _zb(i):
        for j in range(0, ACC_W, 16):
            bounce[i, pl.ds(j, 16)] = zero16

    @pl.loop(0, CHUNK)
    def _zs(i):
        for j in range(0, ACC_W, 16):
            sbuf[i, pl.ds(j, 16)] = zero16

    @pl.loop(0, ROWS_PER_SUB // SLAB)
    def _zacc(j):
        pltpu.sync_copy(bounce, acc.at[pl.ds(sid * ROWS_PER_SUB + j * SLAB, SLAB)])

    plsc.subcore_barrier()

    iota16 = lax.iota(jnp.int32, 16)
    col_w = jnp.full((16,), D, jnp.int32)

    @pl.loop(0, NWIN)
    def _win(w):
        pltpu.sync_copy(src_hbm.at[wid, pl.ds(w * WIN, WIN)], srcw)
        pltpu.sync_copy(dst_hbm.at[wid, pl.ds(w * WIN, WIN)], dstw)

        @pl.loop(0, WIN)
        def _chunk(c):
            pltpu.sync_copy(h_hbm.at[dstw.at[c]], gbuf)
            pltpu.sync_copy(asrc_hbm.at[srcw.at[c]], asbuf)
            pltpu.sync_copy(adst_hbm.at[dstw.at[c]], adbuf)

            for g in range(CHUNK // 16):
                s = asbuf[pl.ds(g * 16, 16)] + adbuf[pl.ds(g * 16, 16)]
                leaky = jnp.where(s > 0, s, ALPHA * s)
                w16 = jnp.exp(-leaky)
                rowids = iota16 + (g * 16)
                plsc.store_scatter(sbuf, [rowids, col_w], w16)
                for r in range(16):
                    row = g * 16 + r
                    wr = _splat_lane(w16, r)
                    for j in range(0, D, 16):
                        sbuf[row, pl.ds(j, 16)] = gbuf[row, pl.ds(j, 16)] * wr

            pltpu.sync_copy(sbuf, acc.at[srcw.at[c]], add=True)

    plsc.subcore_barrier()

    @pl.loop(0, ROWS_PER_SUB // SLAB)
    def _rb(j):
        base = sid * ROWS_PER_SUB + j * SLAB
        pltpu.sync_copy(acc.at[pl.ds(base, SLAB)], bounce)
        pltpu.sync_copy(bounce, out_hbm.at[cid, pl.ds(base, SLAB)])


_edge_kernel = pl.kernel(
    _edge_body,
    out_type=jax.ShapeDtypeStruct((NC, N, ACC_W), _f32),
    mesh=plsc.VectorSubcoreMesh(core_axis_name="c", subcore_axis_name="s"),
    compiler_params=pltpu.CompilerParams(use_tc_tiling_on_sc=False,
                                         needs_layout_passes=False),
    scratch_types=[
        pltpu.VMEM((WIN, CHUNK), jnp.int32),
        pltpu.VMEM((WIN, CHUNK), jnp.int32),
        pltpu.VMEM((CHUNK,), _f32),
        pltpu.VMEM((CHUNK,), _f32),
        pltpu.VMEM((CHUNK, D), _f32),
        pltpu.VMEM((CHUNK, ACC_W), _f32),
        pltpu.VMEM((SLAB, ACC_W), _f32),
        pltpu.VMEM_SHARED((N, ACC_W), _f32),
    ],
)


def _final_body(acc_ref, nn_ref, o_ref):
    a = acc_ref[0] + acc_ref[1]
    num = a[:, :D]
    den = a[:, D:D + 1]
    y = num / den * nn_ref[...]
    o_ref[...] = jnp.where(y > 0, y, jnp.exp(jnp.minimum(y, 0.0)) - 1.0)


_FIN_B = 400


def _final(acc, n_norm):
    grid = N // _FIN_B
    return pl.pallas_call(
        _final_body,
        grid=(grid,),
        in_specs=[
            pl.BlockSpec((NC, _FIN_B, ACC_W), lambda i: (0, i, 0)),
            pl.BlockSpec((_FIN_B, 1), lambda i: (i, 0)),
        ],
        out_specs=pl.BlockSpec((_FIN_B, D), lambda i: (i, 0)),
        out_shape=jax.ShapeDtypeStruct((N, D), _f32),
    )(acc, n_norm)


def kernel(x, edge_index, n_norm, W_fc, W_attn):
    h, asrc, adst = _prep(x, W_fc, W_attn)
    src = edge_index[0].reshape(NW, NCHUNK, CHUNK)
    dst = edge_index[1].reshape(NW, NCHUNK, CHUNK)
    acc = _edge_kernel(h, asrc.reshape(N), adst.reshape(N), src, dst)
    return _final(acc, n_norm)


# speedup vs baseline: 8.8729x; 1.5916x over previous
import functools

import jax
import jax.numpy as jnp
from jax import lax
from jax.experimental import pallas as pl
from jax.experimental.pallas import tpu as pltpu
from jax.experimental.pallas import tpu_sc as plsc

N = 10000
E = 320000
D = 128
ALPHA = 0.2

NC = 2
NS = 16
NW = NC * NS
EPT = E // NW
CHUNK = 80
NCHUNK = EPT // CHUNK
ROWS_PER_SUB = N // NS
SLAB = 25
ACC_W = 144

_f32 = jnp.float32


def _splat_lane(v, lane):
    dn = lax.GatherDimensionNumbers(
        offset_dims=(), collapsed_slice_dims=(0,), start_index_map=(0,))
    idx = jnp.full((16, 1), lane, jnp.int32)
    return lax.gather(v, idx, dn, (1,),
                      mode=lax.GatherScatterMode.PROMISE_IN_BOUNDS)


def _prep_body(x_ref, wfc_ref, wattn_ref, h_ref, asrc_ref, adst_ref):
    xb = x_ref[...]
    h = lax.dot_general(xb, wfc_ref[...], (((1,), (1,)), ((), ())),
                        preferred_element_type=_f32)
    wa = wattn_ref[...]
    a1 = wa[:, :D]
    a2 = wa[:, D:]
    h_ref[...] = h
    asrc_ref[...] = jnp.sum(h * a1, axis=1, keepdims=True)
    adst_ref[...] = jnp.sum(h * a2, axis=1, keepdims=True)


_PREP_B = 400


def _prep(x, W_fc, W_attn):
    grid = N // _PREP_B
    return pl.pallas_call(
        _prep_body,
        grid=(grid,),
        in_specs=[
            pl.BlockSpec((_PREP_B, D), lambda i: (i, 0)),
            pl.BlockSpec((D, D), lambda i: (0, 0)),
            pl.BlockSpec((1, 2 * D), lambda i: (0, 0)),
        ],
        out_specs=[
            pl.BlockSpec((_PREP_B, D), lambda i: (i, 0)),
            pl.BlockSpec((_PREP_B, 1), lambda i: (i, 0)),
            pl.BlockSpec((_PREP_B, 1), lambda i: (i, 0)),
        ],
        out_shape=[
            jax.ShapeDtypeStruct((N, D), _f32),
            jax.ShapeDtypeStruct((N, 1), _f32),
            jax.ShapeDtypeStruct((N, 1), _f32),
        ],
    )(x, W_fc, W_attn)


WIN = 25
NWIN = NCHUNK // WIN


def _edge_body(h_hbm, asrc_hbm, adst_hbm, src_hbm, dst_hbm, out_hbm,
               srcw, dstw, asbuf, adbuf, gbuf, sbuf, bounce, gsem, ssem, acc):
    cid = lax.axis_index("c")
    sid = lax.axis_index("s")
    wid = cid * NS + sid

    zero16 = jnp.zeros((16,), _f32)

    @pl.loop(0, SLAB)
    def _zb(i):
        for j in range(0, ACC_W, 16):
            bounce[i, pl.ds(j, 16)] = zero16

    @pl.loop(0, CHUNK)
    def _zs(i):
        for j in range(0, ACC_W, 16):
            sbuf[i, pl.ds(j, 16)] = zero16

    @pl.loop(0, ROWS_PER_SUB // SLAB)
    def _zacc(j):
        pltpu.sync_copy(bounce, acc.at[pl.ds(sid * ROWS_PER_SUB + j * SLAB, SLAB)])

    plsc.subcore_barrier()

    iota16 = lax.iota(jnp.int32, 16)
    col_w = jnp.full((16,), D, jnp.int32)

    @pl.loop(0, NWIN)
    def _win(w):
        @pl.when(w > 0)
        def _():
            pltpu.make_async_copy(sbuf, acc.at[srcw.at[0]], ssem).wait()

        pltpu.sync_copy(src_hbm.at[wid, pl.ds(w * WIN, WIN)], srcw)
        pltpu.sync_copy(dst_hbm.at[wid, pl.ds(w * WIN, WIN)], dstw)

        @pl.loop(0, WIN)
        def _chunk(c):
            g1 = pltpu.async_copy(h_hbm.at[dstw.at[c]], gbuf, gsem)
            g2 = pltpu.async_copy(asrc_hbm.at[srcw.at[c]], asbuf, gsem)
            g3 = pltpu.async_copy(adst_hbm.at[dstw.at[c]], adbuf, gsem)
            g1.wait()
            g2.wait()
            g3.wait()

            @pl.when(c > 0)
            def _():
                pltpu.make_async_copy(sbuf, acc.at[srcw.at[c]], ssem).wait()

            for g in range(CHUNK // 16):
                s = asbuf[pl.ds(g * 16, 16)] + adbuf[pl.ds(g * 16, 16)]
                leaky = jnp.where(s > 0, s, ALPHA * s)
                w16 = jnp.exp(-leaky)
                rowids = iota16 + (g * 16)
                plsc.store_scatter(sbuf, [rowids, col_w], w16)
                for r in range(16):
                    row = g * 16 + r
                    wr = _splat_lane(w16, r)
                    for j in range(0, D, 16):
                        sbuf[row, pl.ds(j, 16)] = gbuf[row, pl.ds(j, 16)] * wr

            pltpu.async_copy(sbuf, acc.at[srcw.at[c]], ssem, add=True)

    pltpu.make_async_copy(sbuf, acc.at[srcw.at[0]], ssem).wait()

    plsc.subcore_barrier()

    @pl.loop(0, ROWS_PER_SUB // SLAB)
    def _rb(j):
        base = sid * ROWS_PER_SUB + j * SLAB
        pltpu.sync_copy(acc.at[pl.ds(base, SLAB)], bounce)
        pltpu.sync_copy(bounce, out_hbm.at[cid, pl.ds(base, SLAB)])


_edge_kernel = pl.kernel(
    _edge_body,
    out_type=jax.ShapeDtypeStruct((NC, N, ACC_W), _f32),
    mesh=plsc.VectorSubcoreMesh(core_axis_name="c", subcore_axis_name="s"),
    compiler_params=pltpu.CompilerParams(use_tc_tiling_on_sc=False,
                                         needs_layout_passes=False),
    scratch_types=[
        pltpu.VMEM((WIN, CHUNK), jnp.int32),
        pltpu.VMEM((WIN, CHUNK), jnp.int32),
        pltpu.VMEM((CHUNK,), _f32),
        pltpu.VMEM((CHUNK,), _f32),
        pltpu.VMEM((CHUNK, D), _f32),
        pltpu.VMEM((CHUNK, ACC_W), _f32),
        pltpu.VMEM((SLAB, ACC_W), _f32),
        pltpu.SemaphoreType.DMA,
        pltpu.SemaphoreType.DMA,
        pltpu.VMEM_SHARED((N, ACC_W), _f32),
    ],
)


def _final_body(acc_ref, nn_ref, o_ref):
    a = acc_ref[0] + acc_ref[1]
    num = a[:, :D]
    den = a[:, D:D + 1]
    y = num / den * nn_ref[...]
    o_ref[...] = jnp.where(y > 0, y, jnp.exp(jnp.minimum(y, 0.0)) - 1.0)


_FIN_B = 400


def _final(acc, n_norm):
    grid = N // _FIN_B
    return pl.pallas_call(
        _final_body,
        grid=(grid,),
        in_specs=[
            pl.BlockSpec((NC, _FIN_B, ACC_W), lambda i: (0, i, 0)),
            pl.BlockSpec((_FIN_B, 1), lambda i: (i, 0)),
        ],
        out_specs=pl.BlockSpec((_FIN_B, D), lambda i: (i, 0)),
        out_shape=jax.ShapeDtypeStruct((N, D), _f32),
    )(acc, n_norm)


def kernel(x, edge_index, n_norm, W_fc, W_attn):
    h, asrc, adst = _prep(x, W_fc, W_attn)
    src = edge_index[0].reshape(NW, NCHUNK, CHUNK)
    dst = edge_index[1].reshape(NW, NCHUNK, CHUNK)
    acc = _edge_kernel(h, asrc.reshape(N), adst.reshape(N), src, dst)
    return _final(acc, n_norm)


# speedup vs baseline: 10.2952x; 1.1603x over previous
import functools

import jax
import jax.numpy as jnp
from jax import lax
from jax.experimental import pallas as pl
from jax.experimental.pallas import tpu as pltpu
from jax.experimental.pallas import tpu_sc as plsc

N = 10000
E = 320000
D = 128
ALPHA = 0.2

NC = 2
NS = 16
NW = NC * NS
EPT = E // NW
CHUNK = 80
NCHUNK = EPT // CHUNK
ROWS_PER_SUB = N // NS
SLAB = 25
ACC_W = 144

_f32 = jnp.float32


def _splat_lane(v, lane):
    dn = lax.GatherDimensionNumbers(
        offset_dims=(), collapsed_slice_dims=(0,), start_index_map=(0,))
    idx = jnp.full((16, 1), lane, jnp.int32)
    return lax.gather(v, idx, dn, (1,),
                      mode=lax.GatherScatterMode.PROMISE_IN_BOUNDS)


def _prep_body(x_ref, wfc_ref, wattn_ref, h_ref, asrc_ref, adst_ref):
    xb = x_ref[...]
    h = lax.dot_general(xb, wfc_ref[...], (((1,), (1,)), ((), ())),
                        preferred_element_type=_f32)
    wa = wattn_ref[...]
    a1 = wa[:, :D]
    a2 = wa[:, D:]
    h_ref[...] = h
    asrc_ref[...] = jnp.sum(h * a1, axis=1, keepdims=True)
    adst_ref[...] = jnp.sum(h * a2, axis=1, keepdims=True)


_PREP_B = 400


def _prep(x, W_fc, W_attn):
    grid = N // _PREP_B
    return pl.pallas_call(
        _prep_body,
        grid=(grid,),
        in_specs=[
            pl.BlockSpec((_PREP_B, D), lambda i: (i, 0)),
            pl.BlockSpec((D, D), lambda i: (0, 0)),
            pl.BlockSpec((1, 2 * D), lambda i: (0, 0)),
        ],
        out_specs=[
            pl.BlockSpec((_PREP_B, D), lambda i: (i, 0)),
            pl.BlockSpec((_PREP_B, 1), lambda i: (i, 0)),
            pl.BlockSpec((_PREP_B, 1), lambda i: (i, 0)),
        ],
        out_shape=[
            jax.ShapeDtypeStruct((N, D), _f32),
            jax.ShapeDtypeStruct((N, 1), _f32),
            jax.ShapeDtypeStruct((N, 1), _f32),
        ],
    )(x, W_fc, W_attn)


WIN = 25
NWIN = NCHUNK // WIN


def _edge_body(h_hbm, asrc_hbm, adst_hbm, src_hbm, dst_hbm, out_hbm,
               srcw, dstw, asbuf0, adbuf0, gbuf0, asbuf1, adbuf1, gbuf1,
               sbuf, bounce, gsem, ssem, acc):
    asbufs, adbufs, gbufs = (asbuf0, asbuf1), (adbuf0, adbuf1), (gbuf0, gbuf1)
    cid = lax.axis_index("c")
    sid = lax.axis_index("s")
    wid = cid * NS + sid

    zero16 = jnp.zeros((16,), _f32)

    @pl.loop(0, SLAB)
    def _zb(i):
        for j in range(0, ACC_W, 16):
            bounce[i, pl.ds(j, 16)] = zero16

    @pl.loop(0, CHUNK)
    def _zs(i):
        for j in range(0, ACC_W, 16):
            sbuf[i, pl.ds(j, 16)] = zero16

    @pl.loop(0, ROWS_PER_SUB // SLAB)
    def _zacc(j):
        pltpu.sync_copy(bounce, acc.at[pl.ds(sid * ROWS_PER_SUB + j * SLAB, SLAB)])

    plsc.subcore_barrier()

    iota16 = lax.iota(jnp.int32, 16)
    col_w = jnp.full((16,), D, jnp.int32)

    def _fire_gathers(c, b):
        pltpu.async_copy(h_hbm.at[dstw.at[c]], gbufs[b], gsem)
        pltpu.async_copy(asrc_hbm.at[srcw.at[c]], asbufs[b], gsem)
        pltpu.async_copy(adst_hbm.at[dstw.at[c]], adbufs[b], gsem)

    def _wait_gathers(c, b):
        pltpu.make_async_copy(h_hbm.at[dstw.at[c]], gbufs[b], gsem).wait()
        pltpu.make_async_copy(asrc_hbm.at[srcw.at[c]], asbufs[b], gsem).wait()
        pltpu.make_async_copy(adst_hbm.at[dstw.at[c]], adbufs[b], gsem).wait()

    @pl.loop(0, NWIN)
    def _win(w):
        @pl.when(w > 0)
        def _():
            pltpu.make_async_copy(sbuf, acc.at[srcw.at[0]], ssem).wait()

        pltpu.sync_copy(src_hbm.at[wid, pl.ds(w * WIN, WIN)], srcw)
        pltpu.sync_copy(dst_hbm.at[wid, pl.ds(w * WIN, WIN)], dstw)

        _fire_gathers(0, 0)

        @pl.loop(0, WIN)
        def _chunk(c):
            par = lax.rem(c, 2)
            for b in range(2):
                @pl.when(par == b)
                def _():
                    _wait_gathers(c, b)

                    @pl.when(c + 1 < WIN)
                    def _():
                        _fire_gathers(c + 1, 1 - b)

                    @pl.when(c > 0)
                    def _():
                        pltpu.make_async_copy(sbuf, acc.at[srcw.at[c]],
                                              ssem).wait()

                    gbuf, asbuf, adbuf = gbufs[b], asbufs[b], adbufs[b]
                    for g in range(CHUNK // 16):
                        s = asbuf[pl.ds(g * 16, 16)] + adbuf[pl.ds(g * 16, 16)]
                        leaky = jnp.where(s > 0, s, ALPHA * s)
                        w16 = jnp.exp(-leaky)
                        rowids = iota16 + (g * 16)
                        plsc.store_scatter(sbuf, [rowids, col_w], w16)
                        for r in range(16):
                            row = g * 16 + r
                            wr = _splat_lane(w16, r)
                            for j in range(0, D, 16):
                                sbuf[row, pl.ds(j, 16)] = (
                                    gbuf[row, pl.ds(j, 16)] * wr)

                    pltpu.async_copy(sbuf, acc.at[srcw.at[c]], ssem, add=True)

    pltpu.make_async_copy(sbuf, acc.at[srcw.at[0]], ssem).wait()

    plsc.subcore_barrier()

    @pl.loop(0, ROWS_PER_SUB // SLAB)
    def _rb(j):
        base = sid * ROWS_PER_SUB + j * SLAB
        pltpu.sync_copy(acc.at[pl.ds(base, SLAB)], bounce)
        pltpu.sync_copy(bounce, out_hbm.at[cid, pl.ds(base, SLAB)])


_edge_kernel = pl.kernel(
    _edge_body,
    out_type=jax.ShapeDtypeStruct((NC, N, ACC_W), _f32),
    mesh=plsc.VectorSubcoreMesh(core_axis_name="c", subcore_axis_name="s"),
    compiler_params=pltpu.CompilerParams(use_tc_tiling_on_sc=False,
                                         needs_layout_passes=False),
    scratch_types=[
        pltpu.VMEM((WIN, CHUNK), jnp.int32),
        pltpu.VMEM((WIN, CHUNK), jnp.int32),
        pltpu.VMEM((CHUNK,), _f32),
        pltpu.VMEM((CHUNK,), _f32),
        pltpu.VMEM((CHUNK, D), _f32),
        pltpu.VMEM((CHUNK,), _f32),
        pltpu.VMEM((CHUNK,), _f32),
        pltpu.VMEM((CHUNK, D), _f32),
        pltpu.VMEM((CHUNK, ACC_W), _f32),
        pltpu.VMEM((SLAB, ACC_W), _f32),
        pltpu.SemaphoreType.DMA,
        pltpu.SemaphoreType.DMA,
        pltpu.VMEM_SHARED((N, ACC_W), _f32),
    ],
)


def _final_body(acc_ref, nn_ref, o_ref):
    a = acc_ref[0] + acc_ref[1]
    num = a[:, :D]
    den = a[:, D:D + 1]
    y = num / den * nn_ref[...]
    o_ref[...] = jnp.where(y > 0, y, jnp.exp(jnp.minimum(y, 0.0)) - 1.0)


_FIN_B = 400


def _final(acc, n_norm):
    grid = N // _FIN_B
    return pl.pallas_call(
        _final_body,
        grid=(grid,),
        in_specs=[
            pl.BlockSpec((NC, _FIN_B, ACC_W), lambda i: (0, i, 0)),
            pl.BlockSpec((_FIN_B, 1), lambda i: (i, 0)),
        ],
        out_specs=pl.BlockSpec((_FIN_B, D), lambda i: (i, 0)),
        out_shape=jax.ShapeDtypeStruct((N, D), _f32),
    )(acc, n_norm)


def kernel(x, edge_index, n_norm, W_fc, W_attn):
    h, asrc, adst = _prep(x, W_fc, W_attn)
    src = edge_index[0].reshape(NW, NCHUNK, CHUNK)
    dst = edge_index[1].reshape(NW, NCHUNK, CHUNK)
    acc = _edge_kernel(h, asrc.reshape(N), adst.reshape(N), src, dst)
    return _final(acc, n_norm)
